# baseline (speedup 1.0000x reference)
import functools

import jax
import jax.numpy as jnp
from jax import lax
from jax.experimental import pallas as pl
from jax.experimental.pallas import tpu as pltpu
from jax.experimental.pallas import tpu_sc as plsc

_N_NODES = 10000
_N_PAD = 10240
_D = 128
_OUT = 64
_NC = 2
_NS = 16
_NW = _NC * _NS
_TILE_ROWS = _N_PAD // _NS
_ROW_BLOCK = 256


def _sc_mesh():
    return plsc.VectorSubcoreMesh(core_axis_name="c", subcore_axis_name="s",
                                  num_cores=_NC, num_subcores=_NS)


def _make_sc_conv(chunks_per_tile, chunk):
    @functools.partial(
        pl.kernel,
        out_type=jax.ShapeDtypeStruct((_NC, _N_PAD, _D), jnp.float32),
        mesh=_sc_mesh(),
        scratch_types=[
            pltpu.VMEM((chunks_per_tile, chunk), jnp.int32),
            pltpu.VMEM((chunks_per_tile, chunk), jnp.int32),
            pltpu.VMEM((chunk, _D), jnp.float32),
            pltpu.VMEM((chunk, _D), jnp.float32),
            pltpu.VMEM_SHARED((_N_PAD, _D), jnp.float32),
            pltpu.SemaphoreType.DMA,
            pltpu.SemaphoreType.DMA,
        ],
    )
    def conv(h_hbm, src_hbm, dst_hbm, zeros_hbm, out_hbm, src_v, dst_v, msg0,
             msg1, acc_sh, sem0, sem1):
        c = lax.axis_index("c")
        s = lax.axis_index("s")
        w = c * _NS + s
        pltpu.sync_copy(src_hbm.at[pl.ds(w * chunks_per_tile, chunks_per_tile)],
                        src_v)
        pltpu.sync_copy(dst_hbm.at[pl.ds(w * chunks_per_tile, chunks_per_tile)],
                        dst_v)
        pltpu.sync_copy(zeros_hbm, msg0)
        for k in range(_TILE_ROWS // chunk):
            pltpu.sync_copy(
                msg0, acc_sh.at[pl.ds(s * _TILE_ROWS + k * chunk, chunk)])
        pltpu.async_copy(h_hbm.at[src_v.at[0]], msg0, sem0)
        plsc.subcore_barrier()

        def wait_dma(buf, sem):
            pltpu.make_async_copy(zeros_hbm, buf, sem).wait()

        @pl.loop(0, chunks_per_tile, step=2)
        def _(j):
            wait_dma(msg0, sem0)
            pltpu.async_copy(h_hbm.at[src_v.at[j + 1]], msg1, sem1)
            pltpu.sync_copy(msg0, acc_sh.at[dst_v.at[j]], add=True)
            wait_dma(msg1, sem1)

            @pl.when(j + 2 < chunks_per_tile)
            def _():
                pltpu.async_copy(h_hbm.at[src_v.at[j + 2]], msg0, sem0)

            pltpu.sync_copy(msg1, acc_sh.at[dst_v.at[j + 1]], add=True)

        plsc.subcore_barrier()
        n_out = _TILE_ROWS // chunk
        for k in range(n_out):
            buf, sem = (msg0, sem0) if k % 2 == 0 else (msg1, sem1)
            if k >= 2:
                pltpu.make_async_copy(zeros_hbm, buf, sem).wait()
            base = s * _TILE_ROWS + k * chunk
            pltpu.sync_copy(acc_sh.at[pl.ds(base, chunk)], buf)
            pltpu.async_copy(buf, out_hbm.at[c, pl.ds(base, chunk)], sem)
        for k in range(max(0, n_out - 2), n_out):
            buf, sem = (msg0, sem0) if k % 2 == 0 else (msg1, sem1)
            pltpu.make_async_copy(zeros_hbm, buf, sem).wait()

    return conv


def _make_sc_deg(chunks_per_tile, chunk):
    @functools.partial(
        pl.kernel,
        out_type=jax.ShapeDtypeStruct((_NC, _N_PAD, _D), jnp.float32),
        mesh=_sc_mesh(),
        scratch_types=[
            pltpu.VMEM((chunks_per_tile, chunk), jnp.int32),
            pltpu.VMEM((chunk, _D), jnp.float32),
            pltpu.VMEM_SHARED((_N_PAD, _D), jnp.float32),
            pltpu.SemaphoreType.DMA,
        ],
    )
    def deg(dst_hbm, ones_hbm, zeros_hbm, out_hbm, dst_v, buf_v, deg_sh, sem):
        c = lax.axis_index("c")
        s = lax.axis_index("s")
        w = c * _NS + s
        pltpu.sync_copy(dst_hbm.at[pl.ds(w * chunks_per_tile, chunks_per_tile)],
                        dst_v)
        pltpu.sync_copy(zeros_hbm, buf_v)
        for k in range(_TILE_ROWS // chunk):
            pltpu.sync_copy(
                buf_v, deg_sh.at[pl.ds(s * _TILE_ROWS + k * chunk, chunk)])
        pltpu.sync_copy(ones_hbm, buf_v)
        plsc.subcore_barrier()

        @pl.loop(0, chunks_per_tile)
        def _(j):
            pltpu.async_copy(buf_v, deg_sh.at[dst_v.at[j]], sem, add=True)

        @pl.loop(0, chunks_per_tile)
        def _(j):
            pltpu.make_async_copy(zeros_hbm, buf_v, sem).wait()

        plsc.subcore_barrier()
        for k in range(_TILE_ROWS // chunk):
            base = s * _TILE_ROWS + k * chunk
            pltpu.sync_copy(deg_sh.at[pl.ds(base, chunk)], buf_v)
            pltpu.sync_copy(buf_v, out_hbm.at[c, pl.ds(base, chunk)])

    return deg


def _tc_conv_first(parts, degparts_c, W, b, Wl0, bl0, Wl1, bl1, Wo, bo):
    n_blocks = _N_PAD // _ROW_BLOCK
    hp = jax.lax.Precision.HIGHEST

    def body(p_ref, dg_ref, w_ref, b_ref, w0_ref, b0_ref, w1_ref, b1_ref,
             wo_ref, bo_ref, o_ref, inv_ref, wc_ref, bc_ref):
        d = dg_ref[0, :, 0:1] + dg_ref[1, :, 0:1]
        inv = 1.0 / jnp.maximum(d, 1.0)
        inv_ref[...] = inv
        p = (p_ref[0] + p_ref[1]) * inv
        h = jax.lax.dot_general(p, w_ref[...], (((1,), (0,)), ((), ())),
                                precision=hp,
                                preferred_element_type=jnp.float32)
        o_ref[...] = jnp.maximum(h + b_ref[...], 0.0)

        @pl.when(pl.program_id(0) == 0)
        def _():
            t = jax.lax.dot_general(w0_ref[...], w1_ref[...],
                                    (((1,), (0,)), ((), ())), precision=hp,
                                    preferred_element_type=jnp.float32)
            wc_ref[...] = jax.lax.dot_general(t, wo_ref[...],
                                              (((1,), (0,)), ((), ())),
                                              precision=hp,
                                              preferred_element_type=jnp.float32)
            u = jax.lax.dot_general(b0_ref[...], w1_ref[...],
                                    (((1,), (0,)), ((), ())), precision=hp,
                                    preferred_element_type=jnp.float32)
            u = u + b1_ref[...]
            bc_ref[...] = jax.lax.dot_general(u, wo_ref[...],
                                              (((1,), (0,)), ((), ())),
                                              precision=hp,
                                              preferred_element_type=jnp.float32
                                              ) + bo_ref[...]

    return pl.pallas_call(
        body,
        grid=(n_blocks,),
        in_specs=[
            pl.BlockSpec((_NC, _ROW_BLOCK, _D), lambda i: (0, i, 0)),
            pl.BlockSpec((_NC, _ROW_BLOCK, _D), lambda i: (0, i, 0)),
            pl.BlockSpec((_D, _D), lambda i: (0, 0)),
            pl.BlockSpec((1, _D), lambda i: (0, 0)),
            pl.BlockSpec((_D, _D), lambda i: (0, 0)),
            pl.BlockSpec((1, _D), lambda i: (0, 0)),
            pl.BlockSpec((_D, _D), lambda i: (0, 0)),
            pl.BlockSpec((1, _D), lambda i: (0, 0)),
            pl.BlockSpec((_D, _OUT), lambda i: (0, 0)),
            pl.BlockSpec((1, _OUT), lambda i: (0, 0)),
        ],
        out_specs=[
            pl.BlockSpec((_ROW_BLOCK, _D), lambda i: (i, 0)),
            pl.BlockSpec((_ROW_BLOCK, 1), lambda i: (i, 0)),
            pl.BlockSpec((_D, _OUT), lambda i: (0, 0)),
            pl.BlockSpec((1, _OUT), lambda i: (0, 0)),
        ],
        out_shape=[
            jax.ShapeDtypeStruct((_N_PAD, _D), jnp.float32),
            jax.ShapeDtypeStruct((_N_PAD, 1), jnp.float32),
            jax.ShapeDtypeStruct((_D, _OUT), jnp.float32),
            jax.ShapeDtypeStruct((1, _OUT), jnp.float32),
        ],
    )(parts, degparts_c, W, b, Wl0, bl0.reshape(1, _D), Wl1,
      bl1.reshape(1, _D), Wo, bo.reshape(1, _OUT))


def _tc_conv_dest(parts, degparts_d, W, b):
    n_blocks = _N_PAD // _ROW_BLOCK

    def body(p_ref, dg_ref, w_ref, b_ref, o_ref):
        d = dg_ref[0, :, 0:1] + dg_ref[1, :, 0:1]
        inv = 1.0 / jnp.maximum(d, 1.0)
        p = (p_ref[0] + p_ref[1]) * inv
        h = jax.lax.dot_general(p, w_ref[...], (((1,), (0,)), ((), ())),
                                precision=jax.lax.Precision.HIGHEST,
                                preferred_element_type=jnp.float32)
        o_ref[...] = jnp.maximum(h + b_ref[...], 0.0)

    return pl.pallas_call(
        body,
        grid=(n_blocks,),
        in_specs=[
            pl.BlockSpec((_NC, _ROW_BLOCK, _D), lambda i: (0, i, 0)),
            pl.BlockSpec((_NC, _ROW_BLOCK, _D), lambda i: (0, i, 0)),
            pl.BlockSpec((_D, _D), lambda i: (0, 0)),
            pl.BlockSpec((1, _D), lambda i: (0, 0)),
        ],
        out_specs=pl.BlockSpec((_ROW_BLOCK, _D), lambda i: (i, 0)),
        out_shape=jax.ShapeDtypeStruct((_N_PAD, _D), jnp.float32),
    )(parts, degparts_d, W, b)


def _tc_conv_update(parts, inv_deg, W, b):
    n_blocks = _N_PAD // _ROW_BLOCK

    def body(p_ref, inv_ref, w_ref, b_ref, o_ref):
        p = (p_ref[0] + p_ref[1]) * inv_ref[...]
        h = jax.lax.dot_general(p, w_ref[...], (((1,), (0,)), ((), ())),
                                precision=jax.lax.Precision.HIGHEST,
                                preferred_element_type=jnp.float32)
        o_ref[...] = jnp.maximum(h + b_ref[...], 0.0)

    return pl.pallas_call(
        body,
        grid=(n_blocks,),
        in_specs=[
            pl.BlockSpec((_NC, _ROW_BLOCK, _D), lambda i: (0, i, 0)),
            pl.BlockSpec((_ROW_BLOCK, 1), lambda i: (i, 0)),
            pl.BlockSpec((_D, _D), lambda i: (0, 0)),
            pl.BlockSpec((1, _D), lambda i: (0, 0)),
        ],
        out_specs=pl.BlockSpec((_ROW_BLOCK, _D), lambda i: (i, 0)),
        out_shape=jax.ShapeDtypeStruct((_N_PAD, _D), jnp.float32),
    )(parts, inv_deg, W, b)


def _tc_conv_head(parts, inv_deg, W, b, Wc, bc):
    n_blocks = _N_PAD // _ROW_BLOCK

    def body(p_ref, inv_ref, w_ref, b_ref, wc_ref, bc_ref, o_ref):
        p = (p_ref[0] + p_ref[1]) * inv_ref[...]
        h = jax.lax.dot_general(p, w_ref[...], (((1,), (0,)), ((), ())),
                                precision=jax.lax.Precision.HIGHEST,
                                preferred_element_type=jnp.float32)
        t = jnp.maximum(h + b_ref[...], 0.0)
        o = jax.lax.dot_general(t, wc_ref[...], (((1,), (0,)), ((), ())),
                                precision=jax.lax.Precision.HIGHEST,
                                preferred_element_type=jnp.float32)
        o_ref[...] = o + bc_ref[...]

    return pl.pallas_call(
        body,
        grid=(n_blocks,),
        in_specs=[
            pl.BlockSpec((_NC, _ROW_BLOCK, _D), lambda i: (0, i, 0)),
            pl.BlockSpec((_ROW_BLOCK, 1), lambda i: (i, 0)),
            pl.BlockSpec((_D, _D), lambda i: (0, 0)),
            pl.BlockSpec((1, _D), lambda i: (0, 0)),
            pl.BlockSpec((_D, _OUT), lambda i: (0, 0)),
            pl.BlockSpec((1, _OUT), lambda i: (0, 0)),
        ],
        out_specs=pl.BlockSpec((_ROW_BLOCK, _OUT), lambda i: (i, 0)),
        out_shape=jax.ShapeDtypeStruct((_N_PAD, _OUT), jnp.float32),
    )(parts, inv_deg, W, b, Wc, bc)


def _prep_edges(ei, chunks_per_tile, chunk):
    e = ei.shape[1]
    e_tot = _NW * chunks_per_tile * chunk
    n_pad = e_tot - e
    fill = jnp.arange(n_pad, dtype=jnp.int32)
    pad_src = fill % _N_NODES
    pad_dst = _N_NODES + fill % (_N_PAD - _N_NODES)
    src = jnp.concatenate([ei[0], pad_src]).reshape(_NW * chunks_per_tile,
                                                    chunk)
    dst = jnp.concatenate([ei[1], pad_dst]).reshape(_NW * chunks_per_tile,
                                                    chunk)
    return src, dst


def kernel(x, edge_index_connections, edge_index_destinations, W1, b1, W2, b2,
           W3, b3, W4, b4, Wl0, bl0, Wl1, bl1, Wo, bo):
    cpt_c, chk_c = 40, 128
    cpt_d, chk_d = 32, 80
    src_c, dst_c = _prep_edges(edge_index_connections, cpt_c, chk_c)
    src_d, dst_d = _prep_edges(edge_index_destinations, cpt_d, chk_d)

    x_pad = jnp.concatenate(
        [x[0], jnp.zeros((_N_PAD - _N_NODES, _D), jnp.float32)], axis=0)
    zeros_c = jnp.zeros((chk_c, _D), jnp.float32)
    zeros_d = jnp.zeros((chk_d, _D), jnp.float32)
    ones_c = jnp.ones((chk_c, _D), jnp.float32)
    ones_d = jnp.ones((chk_d, _D), jnp.float32)

    degparts_c = _make_sc_deg(cpt_c, chk_c)(dst_c, ones_c, zeros_c)
    degparts_d = _make_sc_deg(cpt_d, chk_d)(dst_d, ones_d, zeros_d)

    conv_c = _make_sc_conv(cpt_c, chk_c)
    conv_d = _make_sc_conv(cpt_d, chk_d)

    b1r, b2r = b1.reshape(1, _D), b2.reshape(1, _D)
    b3r, b4r = b3.reshape(1, _D), b4.reshape(1, _D)

    h, deg_c, Wc, bc = _tc_conv_first(
        conv_c(x_pad, src_c, dst_c, zeros_c), degparts_c, W1, b1r,
        Wl0, bl0, Wl1, bl1, Wo, bo)
    h = _tc_conv_update(conv_c(h, src_c, dst_c, zeros_c), deg_c, W2, b2r)
    h = _tc_conv_update(conv_c(h, src_c, dst_c, zeros_c), deg_c, W2, b2r)
    h = _tc_conv_dest(conv_d(h, src_d, dst_d, zeros_d), degparts_d, W3, b3r)
    h = _tc_conv_update(conv_c(h, src_c, dst_c, zeros_c), deg_c, W4, b4r)
    out = _tc_conv_head(conv_c(h, src_c, dst_c, zeros_c), deg_c, W4, b4r,
                        Wc, bc)

    out = out[:_N_NODES]
    return (out[None, :, : _OUT // 2], out[None, :, _OUT // 2:])


# speedup vs baseline: 1.0093x; 1.0093x over previous
import functools

import jax
import jax.numpy as jnp
from jax import lax
from jax.experimental import pallas as pl
from jax.experimental.pallas import tpu as pltpu
from jax.experimental.pallas import tpu_sc as plsc

_N_NODES = 10000
_N_PAD = 10240
_D = 128
_OUT = 64
_NC = 2
_NS = 16
_NW = _NC * _NS
_TILE_ROWS = _N_PAD // _NS
_ROW_BLOCK = 256


def _sc_mesh():
    return plsc.VectorSubcoreMesh(core_axis_name="c", subcore_axis_name="s",
                                  num_cores=_NC, num_subcores=_NS)


def _make_sc_conv(chunks_per_tile, chunk):
    @functools.partial(
        pl.kernel,
        out_type=jax.ShapeDtypeStruct((_NC, _N_PAD, _D), jnp.float32),
        mesh=_sc_mesh(),
        scratch_types=[
            pltpu.VMEM((chunks_per_tile, chunk), jnp.int32),
            pltpu.VMEM((chunks_per_tile, chunk), jnp.int32),
            pltpu.VMEM((chunk, _D), jnp.float32),
            pltpu.VMEM((chunk, _D), jnp.float32),
            pltpu.VMEM_SHARED((_N_PAD, _D), jnp.float32),
            pltpu.SemaphoreType.DMA,
            pltpu.SemaphoreType.DMA,
        ],
    )
    def conv(h_hbm, src_hbm, dst_hbm, zeros_hbm, out_hbm, src_v, dst_v, msg0,
             msg1, acc_sh, sem0, sem1):
        c = lax.axis_index("c")
        s = lax.axis_index("s")
        w = c * _NS + s
        pltpu.sync_copy(src_hbm.at[pl.ds(w * chunks_per_tile, chunks_per_tile)],
                        src_v)
        pltpu.sync_copy(dst_hbm.at[pl.ds(w * chunks_per_tile, chunks_per_tile)],
                        dst_v)
        pltpu.sync_copy(zeros_hbm, msg0)
        for k in range(_TILE_ROWS // chunk):
            pltpu.sync_copy(
                msg0, acc_sh.at[pl.ds(s * _TILE_ROWS + k * chunk, chunk)])
        pltpu.async_copy(h_hbm.at[src_v.at[0]], msg0, sem0)
        plsc.subcore_barrier()

        def wait_dma(buf, sem):
            pltpu.make_async_copy(zeros_hbm, buf, sem).wait()

        @pl.loop(0, chunks_per_tile, step=2)
        def _(j):
            wait_dma(msg0, sem0)
            pltpu.async_copy(h_hbm.at[src_v.at[j + 1]], msg1, sem1)
            pltpu.sync_copy(msg0, acc_sh.at[dst_v.at[j]], add=True)
            wait_dma(msg1, sem1)

            @pl.when(j + 2 < chunks_per_tile)
            def _():
                pltpu.async_copy(h_hbm.at[src_v.at[j + 2]], msg0, sem0)

            pltpu.sync_copy(msg1, acc_sh.at[dst_v.at[j + 1]], add=True)

        plsc.subcore_barrier()
        n_out = _TILE_ROWS // chunk
        for k in range(n_out):
            buf, sem = (msg0, sem0) if k % 2 == 0 else (msg1, sem1)
            if k >= 2:
                pltpu.make_async_copy(zeros_hbm, buf, sem).wait()
            base = s * _TILE_ROWS + k * chunk
            pltpu.sync_copy(acc_sh.at[pl.ds(base, chunk)], buf)
            pltpu.async_copy(buf, out_hbm.at[c, pl.ds(base, chunk)], sem)
        for k in range(max(0, n_out - 2), n_out):
            buf, sem = (msg0, sem0) if k % 2 == 0 else (msg1, sem1)
            pltpu.make_async_copy(zeros_hbm, buf, sem).wait()

    return conv


def _make_sc_deg(chunks_per_tile, chunk):
    @functools.partial(
        pl.kernel,
        out_type=jax.ShapeDtypeStruct((_NC, _N_PAD, _D), jnp.float32),
        mesh=_sc_mesh(),
        scratch_types=[
            pltpu.VMEM((chunks_per_tile, chunk), jnp.int32),
            pltpu.VMEM((chunk, _D), jnp.float32),
            pltpu.VMEM_SHARED((_N_PAD, _D), jnp.float32),
            pltpu.SemaphoreType.DMA,
        ],
    )
    def deg(dst_hbm, ones_hbm, zeros_hbm, out_hbm, dst_v, buf_v, deg_sh, sem):
        c = lax.axis_index("c")
        s = lax.axis_index("s")
        w = c * _NS + s
        pltpu.sync_copy(dst_hbm.at[pl.ds(w * chunks_per_tile, chunks_per_tile)],
                        dst_v)
        pltpu.sync_copy(zeros_hbm, buf_v)
        for k in range(_TILE_ROWS // chunk):
            pltpu.sync_copy(
                buf_v, deg_sh.at[pl.ds(s * _TILE_ROWS + k * chunk, chunk)])
        pltpu.sync_copy(ones_hbm, buf_v)
        plsc.subcore_barrier()

        @pl.loop(0, chunks_per_tile)
        def _(j):
            pltpu.async_copy(buf_v, deg_sh.at[dst_v.at[j]], sem, add=True)

        @pl.loop(0, chunks_per_tile)
        def _(j):
            pltpu.make_async_copy(zeros_hbm, buf_v, sem).wait()

        plsc.subcore_barrier()
        for k in range(_TILE_ROWS // chunk):
            base = s * _TILE_ROWS + k * chunk
            pltpu.sync_copy(deg_sh.at[pl.ds(base, chunk)], buf_v)
            pltpu.sync_copy(buf_v, out_hbm.at[c, pl.ds(base, chunk)])

    return deg


def _tc_prep_c(degparts_c, Wl0, bl0, Wl1, bl1, Wo, bo):
    n_blocks = _N_PAD // _ROW_BLOCK
    hp = jax.lax.Precision.HIGHEST

    def body(dgc_ref, w0_ref, b0_ref, w1_ref, b1_ref, wo_ref, bo_ref,
             invc_ref, wc_ref, bc_ref):
        dc = dgc_ref[0, :, 0:1] + dgc_ref[1, :, 0:1]
        invc_ref[...] = 1.0 / jnp.maximum(dc, 1.0)

        @pl.when(pl.program_id(0) == 0)
        def _():
            t = jax.lax.dot_general(w0_ref[...], w1_ref[...],
                                    (((1,), (0,)), ((), ())), precision=hp,
                                    preferred_element_type=jnp.float32)
            wc_ref[...] = jax.lax.dot_general(t, wo_ref[...],
                                              (((1,), (0,)), ((), ())),
                                              precision=hp,
                                              preferred_element_type=jnp.float32)
            u = jax.lax.dot_general(b0_ref[...], w1_ref[...],
                                    (((1,), (0,)), ((), ())), precision=hp,
                                    preferred_element_type=jnp.float32)
            u = u + b1_ref[...]
            bc_ref[...] = jax.lax.dot_general(u, wo_ref[...],
                                              (((1,), (0,)), ((), ())),
                                              precision=hp,
                                              preferred_element_type=jnp.float32
                                              ) + bo_ref[...]

    return pl.pallas_call(
        body,
        grid=(n_blocks,),
        in_specs=[
            pl.BlockSpec((_NC, _ROW_BLOCK, _D), lambda i: (0, i, 0)),
            pl.BlockSpec((_D, _D), lambda i: (0, 0)),
            pl.BlockSpec((1, _D), lambda i: (0, 0)),
            pl.BlockSpec((_D, _D), lambda i: (0, 0)),
            pl.BlockSpec((1, _D), lambda i: (0, 0)),
            pl.BlockSpec((_D, _OUT), lambda i: (0, 0)),
            pl.BlockSpec((1, _OUT), lambda i: (0, 0)),
        ],
        out_specs=[
            pl.BlockSpec((_ROW_BLOCK, 1), lambda i: (i, 0)),
            pl.BlockSpec((_D, _OUT), lambda i: (0, 0)),
            pl.BlockSpec((1, _OUT), lambda i: (0, 0)),
        ],
        out_shape=[
            jax.ShapeDtypeStruct((_N_PAD, 1), jnp.float32),
            jax.ShapeDtypeStruct((_D, _OUT), jnp.float32),
            jax.ShapeDtypeStruct((1, _OUT), jnp.float32),
        ],
    )(degparts_c, Wl0, bl0.reshape(1, _D), Wl1, bl1.reshape(1, _D), Wo,
      bo.reshape(1, _OUT))


def _tc_inv_deg(degparts):
    n_blocks = _N_PAD // _ROW_BLOCK

    def body(dg_ref, o_ref):
        d = dg_ref[0, :, 0:1] + dg_ref[1, :, 0:1]
        o_ref[...] = 1.0 / jnp.maximum(d, 1.0)

    return pl.pallas_call(
        body,
        grid=(n_blocks,),
        in_specs=[pl.BlockSpec((_NC, _ROW_BLOCK, _D), lambda i: (0, i, 0))],
        out_specs=pl.BlockSpec((_ROW_BLOCK, 1), lambda i: (i, 0)),
        out_shape=jax.ShapeDtypeStruct((_N_PAD, 1), jnp.float32),
    )(degparts)


def _tc_conv_update(parts, inv_deg, W, b):
    n_blocks = _N_PAD // _ROW_BLOCK

    def body(p_ref, inv_ref, w_ref, b_ref, o_ref):
        p = (p_ref[0] + p_ref[1]) * inv_ref[...]
        h = jax.lax.dot_general(p, w_ref[...], (((1,), (0,)), ((), ())),
                                precision=jax.lax.Precision.HIGHEST,
                                preferred_element_type=jnp.float32)
        o_ref[...] = jnp.maximum(h + b_ref[...], 0.0)

    return pl.pallas_call(
        body,
        grid=(n_blocks,),
        in_specs=[
            pl.BlockSpec((_NC, _ROW_BLOCK, _D), lambda i: (0, i, 0)),
            pl.BlockSpec((_ROW_BLOCK, 1), lambda i: (i, 0)),
            pl.BlockSpec((_D, _D), lambda i: (0, 0)),
            pl.BlockSpec((1, _D), lambda i: (0, 0)),
        ],
        out_specs=pl.BlockSpec((_ROW_BLOCK, _D), lambda i: (i, 0)),
        out_shape=jax.ShapeDtypeStruct((_N_PAD, _D), jnp.float32),
    )(parts, inv_deg, W, b)


def _tc_conv_head(parts, inv_deg, W, b, Wc, bc):
    n_blocks = _N_PAD // _ROW_BLOCK

    def body(p_ref, inv_ref, w_ref, b_ref, wc_ref, bc_ref, o_ref):
        p = (p_ref[0] + p_ref[1]) * inv_ref[...]
        h = jax.lax.dot_general(p, w_ref[...], (((1,), (0,)), ((), ())),
                                precision=jax.lax.Precision.HIGHEST,
                                preferred_element_type=jnp.float32)
        t = jnp.maximum(h + b_ref[...], 0.0)
        o = jax.lax.dot_general(t, wc_ref[...], (((1,), (0,)), ((), ())),
                                precision=jax.lax.Precision.HIGHEST,
                                preferred_element_type=jnp.float32)
        o_ref[...] = o + bc_ref[...]

    return pl.pallas_call(
        body,
        grid=(n_blocks,),
        in_specs=[
            pl.BlockSpec((_NC, _ROW_BLOCK, _D), lambda i: (0, i, 0)),
            pl.BlockSpec((_ROW_BLOCK, 1), lambda i: (i, 0)),
            pl.BlockSpec((_D, _D), lambda i: (0, 0)),
            pl.BlockSpec((1, _D), lambda i: (0, 0)),
            pl.BlockSpec((_D, _OUT), lambda i: (0, 0)),
            pl.BlockSpec((1, _OUT), lambda i: (0, 0)),
        ],
        out_specs=pl.BlockSpec((_ROW_BLOCK, _OUT), lambda i: (i, 0)),
        out_shape=jax.ShapeDtypeStruct((_N_PAD, _OUT), jnp.float32),
    )(parts, inv_deg, W, b, Wc, bc)


def _prep_edges(ei, chunks_per_tile, chunk):
    e = ei.shape[1]
    e_tot = _NW * chunks_per_tile * chunk
    n_pad = e_tot - e
    fill = jnp.arange(n_pad, dtype=jnp.int32)
    pad_src = fill % _N_NODES
    pad_dst = _N_NODES + fill % (_N_PAD - _N_NODES)
    src = jnp.concatenate([ei[0], pad_src]).reshape(_NW * chunks_per_tile,
                                                    chunk)
    dst = jnp.concatenate([ei[1], pad_dst]).reshape(_NW * chunks_per_tile,
                                                    chunk)
    return src, dst


def kernel(x, edge_index_connections, edge_index_destinations, W1, b1, W2, b2,
           W3, b3, W4, b4, Wl0, bl0, Wl1, bl1, Wo, bo):
    cpt_c, chk_c = 40, 128
    cpt_d, chk_d = 32, 80
    src_c, dst_c = _prep_edges(edge_index_connections, cpt_c, chk_c)
    src_d, dst_d = _prep_edges(edge_index_destinations, cpt_d, chk_d)

    x_pad = jnp.concatenate(
        [x[0], jnp.zeros((_N_PAD - _N_NODES, _D), jnp.float32)], axis=0)
    zeros_c = jnp.zeros((chk_c, _D), jnp.float32)
    zeros_d = jnp.zeros((chk_d, _D), jnp.float32)
    ones_c = jnp.ones((chk_c, _D), jnp.float32)
    ones_d = jnp.ones((chk_d, _D), jnp.float32)

    degparts_c = _make_sc_deg(cpt_c, chk_c)(dst_c, ones_c, zeros_c)
    degparts_d = _make_sc_deg(cpt_d, chk_d)(dst_d, ones_d, zeros_d)
    deg_c, Wc, bc = _tc_prep_c(degparts_c, Wl0, bl0, Wl1, bl1, Wo, bo)
    deg_d = _tc_inv_deg(degparts_d)

    conv_c = _make_sc_conv(cpt_c, chk_c)
    conv_d = _make_sc_conv(cpt_d, chk_d)

    b1r, b2r = b1.reshape(1, _D), b2.reshape(1, _D)
    b3r, b4r = b3.reshape(1, _D), b4.reshape(1, _D)

    h = _tc_conv_update(conv_c(x_pad, src_c, dst_c, zeros_c), deg_c, W1, b1r)
    h = _tc_conv_update(conv_c(h, src_c, dst_c, zeros_c), deg_c, W2, b2r)
    h = _tc_conv_update(conv_c(h, src_c, dst_c, zeros_c), deg_c, W2, b2r)
    h = _tc_conv_update(conv_d(h, src_d, dst_d, zeros_d), deg_d, W3, b3r)
    h = _tc_conv_update(conv_c(h, src_c, dst_c, zeros_c), deg_c, W4, b4r)
    out = _tc_conv_head(conv_c(h, src_c, dst_c, zeros_c), deg_c, W4, b4r,
                        Wc, bc)

    out = out[:_N_NODES]
    return (out[None, :, : _OUT // 2], out[None, :, _OUT // 2:])


# speedup vs baseline: 1.1225x; 1.1122x over previous
import functools

import jax
import jax.numpy as jnp
from jax import lax
from jax.experimental import pallas as pl
from jax.experimental.pallas import tpu as pltpu
from jax.experimental.pallas import tpu_sc as plsc

_N_NODES = 10000
_N_PAD = 10240
_D = 128
_OUT = 64
_NC = 2
_NS = 16
_NW = _NC * _NS
_TILE_ROWS = _N_PAD // _NS
_ROW_BLOCK = 512


def _sc_mesh():
    return plsc.VectorSubcoreMesh(core_axis_name="c", subcore_axis_name="s",
                                  num_cores=_NC, num_subcores=_NS)


def _make_sc_conv(chunks_per_tile, chunk):
    @functools.partial(
        pl.kernel,
        out_type=jax.ShapeDtypeStruct((_NC, _N_PAD, _D), jnp.float32),
        mesh=_sc_mesh(),
        scratch_types=[
            pltpu.VMEM((chunks_per_tile, chunk), jnp.int32),
            pltpu.VMEM((chunks_per_tile, chunk), jnp.int32),
            pltpu.VMEM((chunk, _D), jnp.float32),
            pltpu.VMEM((chunk, _D), jnp.float32),
            pltpu.VMEM_SHARED((_N_PAD, _D), jnp.float32),
            pltpu.SemaphoreType.DMA,
            pltpu.SemaphoreType.DMA,
        ],
    )
    def conv(h_hbm, src_hbm, dst_hbm, zeros_hbm, out_hbm, src_v, dst_v, msg0,
             msg1, acc_sh, sem0, sem1):
        c = lax.axis_index("c")
        s = lax.axis_index("s")
        w = c * _NS + s
        pltpu.sync_copy(src_hbm.at[pl.ds(w * chunks_per_tile, chunks_per_tile)],
                        src_v)
        pltpu.sync_copy(dst_hbm.at[pl.ds(w * chunks_per_tile, chunks_per_tile)],
                        dst_v)
        pltpu.sync_copy(zeros_hbm, msg0)
        for k in range(_TILE_ROWS // chunk):
            pltpu.sync_copy(
                msg0, acc_sh.at[pl.ds(s * _TILE_ROWS + k * chunk, chunk)])
        pltpu.async_copy(h_hbm.at[src_v.at[0]], msg0, sem0)
        plsc.subcore_barrier()

        def wait_dma(buf, sem):
            pltpu.make_async_copy(zeros_hbm, buf, sem).wait()

        @pl.loop(0, chunks_per_tile, step=2)
        def _(j):
            wait_dma(msg0, sem0)
            pltpu.async_copy(h_hbm.at[src_v.at[j + 1]], msg1, sem1)
            pltpu.sync_copy(msg0, acc_sh.at[dst_v.at[j]], add=True)
            wait_dma(msg1, sem1)

            @pl.when(j + 2 < chunks_per_tile)
            def _():
                pltpu.async_copy(h_hbm.at[src_v.at[j + 2]], msg0, sem0)

            pltpu.sync_copy(msg1, acc_sh.at[dst_v.at[j + 1]], add=True)

        plsc.subcore_barrier()
        n_out = _TILE_ROWS // chunk
        for k in range(n_out):
            buf, sem = (msg0, sem0) if k % 2 == 0 else (msg1, sem1)
            if k >= 2:
                pltpu.make_async_copy(zeros_hbm, buf, sem).wait()
            base = s * _TILE_ROWS + k * chunk
            pltpu.sync_copy(acc_sh.at[pl.ds(base, chunk)], buf)
            pltpu.async_copy(buf, out_hbm.at[c, pl.ds(base, chunk)], sem)
        for k in range(max(0, n_out - 2), n_out):
            buf, sem = (msg0, sem0) if k % 2 == 0 else (msg1, sem1)
            pltpu.make_async_copy(zeros_hbm, buf, sem).wait()

    return conv


def _make_sc_deg(chunks_per_tile, chunk, width=_D):
    @functools.partial(
        pl.kernel,
        out_type=jax.ShapeDtypeStruct((_NC, _N_PAD, width), jnp.float32),
        mesh=_sc_mesh(),
        scratch_types=[
            pltpu.VMEM((chunks_per_tile, chunk), jnp.int32),
            pltpu.VMEM((chunk, width), jnp.float32),
            pltpu.VMEM_SHARED((_N_PAD, width), jnp.float32),
            pltpu.SemaphoreType.DMA,
        ],
    )
    def deg(dst_hbm, ones_hbm, zeros_hbm, out_hbm, dst_v, buf_v, deg_sh, sem):
        c = lax.axis_index("c")
        s = lax.axis_index("s")
        w = c * _NS + s
        pltpu.sync_copy(dst_hbm.at[pl.ds(w * chunks_per_tile, chunks_per_tile)],
                        dst_v)
        pltpu.sync_copy(zeros_hbm, buf_v)
        for k in range(_TILE_ROWS // chunk):
            pltpu.sync_copy(
                buf_v, deg_sh.at[pl.ds(s * _TILE_ROWS + k * chunk, chunk)])
        pltpu.sync_copy(ones_hbm, buf_v)
        plsc.subcore_barrier()

        @pl.loop(0, chunks_per_tile)
        def _(j):
            pltpu.async_copy(buf_v, deg_sh.at[dst_v.at[j]], sem, add=True)

        @pl.loop(0, chunks_per_tile)
        def _(j):
            pltpu.make_async_copy(zeros_hbm, buf_v, sem).wait()

        plsc.subcore_barrier()
        for k in range(_TILE_ROWS // chunk):
            base = s * _TILE_ROWS + k * chunk
            pltpu.sync_copy(deg_sh.at[pl.ds(base, chunk)], buf_v)
            pltpu.sync_copy(buf_v, out_hbm.at[c, pl.ds(base, chunk)])

    return deg


def _tc_prep_c(degparts_c, Wl0, bl0, Wl1, bl1, Wo, bo):
    n_blocks = _N_PAD // _ROW_BLOCK
    hp = jax.lax.Precision.HIGHEST

    def body(dgc_ref, w0_ref, b0_ref, w1_ref, b1_ref, wo_ref, bo_ref,
             invc_ref, wc_ref, bc_ref):
        dc = dgc_ref[0, :, 0:1] + dgc_ref[1, :, 0:1]
        invc_ref[...] = 1.0 / jnp.maximum(dc, 1.0)

        @pl.when(pl.program_id(0) == 0)
        def _():
            t = jax.lax.dot_general(w0_ref[...], w1_ref[...],
                                    (((1,), (0,)), ((), ())), precision=hp,
                                    preferred_element_type=jnp.float32)
            wc_ref[...] = jax.lax.dot_general(t, wo_ref[...],
                                              (((1,), (0,)), ((), ())),
                                              precision=hp,
                                              preferred_element_type=jnp.float32)
            u = jax.lax.dot_general(b0_ref[...], w1_ref[...],
                                    (((1,), (0,)), ((), ())), precision=hp,
                                    preferred_element_type=jnp.float32)
            u = u + b1_ref[...]
            bc_ref[...] = jax.lax.dot_general(u, wo_ref[...],
                                              (((1,), (0,)), ((), ())),
                                              precision=hp,
                                              preferred_element_type=jnp.float32
                                              ) + bo_ref[...]

    return pl.pallas_call(
        body,
        grid=(n_blocks,),
        in_specs=[
            pl.BlockSpec((_NC, _ROW_BLOCK, _D), lambda i: (0, i, 0)),
            pl.BlockSpec((_D, _D), lambda i: (0, 0)),
            pl.BlockSpec((1, _D), lambda i: (0, 0)),
            pl.BlockSpec((_D, _D), lambda i: (0, 0)),
            pl.BlockSpec((1, _D), lambda i: (0, 0)),
            pl.BlockSpec((_D, _OUT), lambda i: (0, 0)),
            pl.BlockSpec((1, _OUT), lambda i: (0, 0)),
        ],
        out_specs=[
            pl.BlockSpec((_ROW_BLOCK, 1), lambda i: (i, 0)),
            pl.BlockSpec((_D, _OUT), lambda i: (0, 0)),
            pl.BlockSpec((1, _OUT), lambda i: (0, 0)),
        ],
        out_shape=[
            jax.ShapeDtypeStruct((_N_PAD, 1), jnp.float32),
            jax.ShapeDtypeStruct((_D, _OUT), jnp.float32),
            jax.ShapeDtypeStruct((1, _OUT), jnp.float32),
        ],
    )(degparts_c, Wl0, bl0.reshape(1, _D), Wl1, bl1.reshape(1, _D), Wo,
      bo.reshape(1, _OUT))


def _tc_inv_deg(degparts):
    n_blocks = _N_PAD // _ROW_BLOCK

    def body(dg_ref, o_ref):
        d = dg_ref[0, :, 0:1] + dg_ref[1, :, 0:1]
        o_ref[...] = 1.0 / jnp.maximum(d, 1.0)

    return pl.pallas_call(
        body,
        grid=(n_blocks,),
        in_specs=[pl.BlockSpec((_NC, _ROW_BLOCK, _D), lambda i: (0, i, 0))],
        out_specs=pl.BlockSpec((_ROW_BLOCK, 1), lambda i: (i, 0)),
        out_shape=jax.ShapeDtypeStruct((_N_PAD, 1), jnp.float32),
    )(degparts)


def _tc_conv_update(parts, inv_deg, W, b):
    n_blocks = _N_PAD // _ROW_BLOCK

    def body(p_ref, inv_ref, w_ref, b_ref, o_ref):
        p = (p_ref[0] + p_ref[1]) * inv_ref[...]
        h = jax.lax.dot_general(p, w_ref[...], (((1,), (0,)), ((), ())),
                                precision=jax.lax.Precision.DEFAULT,
                                preferred_element_type=jnp.float32)
        o_ref[...] = jnp.maximum(h + b_ref[...], 0.0)

    return pl.pallas_call(
        body,
        grid=(n_blocks,),
        in_specs=[
            pl.BlockSpec((_NC, _ROW_BLOCK, _D), lambda i: (0, i, 0)),
            pl.BlockSpec((_ROW_BLOCK, 1), lambda i: (i, 0)),
            pl.BlockSpec((_D, _D), lambda i: (0, 0)),
            pl.BlockSpec((1, _D), lambda i: (0, 0)),
        ],
        out_specs=pl.BlockSpec((_ROW_BLOCK, _D), lambda i: (i, 0)),
        out_shape=jax.ShapeDtypeStruct((_N_PAD, _D), jnp.float32),
    )(parts, inv_deg, W, b)


def _tc_conv_head(parts, inv_deg, W, b, Wc, bc):
    n_blocks = _N_PAD // _ROW_BLOCK

    def body(p_ref, inv_ref, w_ref, b_ref, wc_ref, bc_ref, o_ref):
        p = (p_ref[0] + p_ref[1]) * inv_ref[...]
        h = jax.lax.dot_general(p, w_ref[...], (((1,), (0,)), ((), ())),
                                precision=jax.lax.Precision.DEFAULT,
                                preferred_element_type=jnp.float32)
        t = jnp.maximum(h + b_ref[...], 0.0)
        o = jax.lax.dot_general(t, wc_ref[...], (((1,), (0,)), ((), ())),
                                precision=jax.lax.Precision.DEFAULT,
                                preferred_element_type=jnp.float32)
        o_ref[...] = o + bc_ref[...]

    return pl.pallas_call(
        body,
        grid=(n_blocks,),
        in_specs=[
            pl.BlockSpec((_NC, _ROW_BLOCK, _D), lambda i: (0, i, 0)),
            pl.BlockSpec((_ROW_BLOCK, 1), lambda i: (i, 0)),
            pl.BlockSpec((_D, _D), lambda i: (0, 0)),
            pl.BlockSpec((1, _D), lambda i: (0, 0)),
            pl.BlockSpec((_D, _OUT), lambda i: (0, 0)),
            pl.BlockSpec((1, _OUT), lambda i: (0, 0)),
        ],
        out_specs=pl.BlockSpec((_ROW_BLOCK, _OUT), lambda i: (i, 0)),
        out_shape=jax.ShapeDtypeStruct((_N_PAD, _OUT), jnp.float32),
    )(parts, inv_deg, W, b, Wc, bc)


def _prep_edges(ei, chunks_per_tile, chunk):
    e = ei.shape[1]
    e_tot = _NW * chunks_per_tile * chunk
    n_pad = e_tot - e
    fill = jnp.arange(n_pad, dtype=jnp.int32)
    pad_src = fill % _N_NODES
    pad_dst = _N_NODES + fill % (_N_PAD - _N_NODES)
    src = jnp.concatenate([ei[0], pad_src]).reshape(_NW * chunks_per_tile,
                                                    chunk)
    dst = jnp.concatenate([ei[1], pad_dst]).reshape(_NW * chunks_per_tile,
                                                    chunk)
    return src, dst


def kernel(x, edge_index_connections, edge_index_destinations, W1, b1, W2, b2,
           W3, b3, W4, b4, Wl0, bl0, Wl1, bl1, Wo, bo):
    cpt_c, chk_c = 40, 128
    cpt_d, chk_d = 32, 80
    src_c, dst_c = _prep_edges(edge_index_connections, cpt_c, chk_c)
    src_d, dst_d = _prep_edges(edge_index_destinations, cpt_d, chk_d)

    x_pad = jnp.concatenate(
        [x[0], jnp.zeros((_N_PAD - _N_NODES, _D), jnp.float32)], axis=0)
    zeros_c = jnp.zeros((chk_c, _D), jnp.float32)
    zeros_d = jnp.zeros((chk_d, _D), jnp.float32)
    ones_c = jnp.ones((chk_c, _D), jnp.float32)
    ones_d = jnp.ones((chk_d, _D), jnp.float32)

    degparts_c = _make_sc_deg(cpt_c, chk_c)(dst_c, ones_c, zeros_c)
    degparts_d = _make_sc_deg(cpt_d, chk_d)(dst_d, ones_d, zeros_d)
    deg_c, Wc, bc = _tc_prep_c(degparts_c, Wl0, bl0, Wl1, bl1, Wo, bo)
    deg_d = _tc_inv_deg(degparts_d)

    conv_c = _make_sc_conv(cpt_c, chk_c)
    conv_d = _make_sc_conv(cpt_d, chk_d)

    b1r, b2r = b1.reshape(1, _D), b2.reshape(1, _D)
    b3r, b4r = b3.reshape(1, _D), b4.reshape(1, _D)

    h = _tc_conv_update(conv_c(x_pad, src_c, dst_c, zeros_c), deg_c, W1, b1r)
    h = _tc_conv_update(conv_c(h, src_c, dst_c, zeros_c), deg_c, W2, b2r)
    h = _tc_conv_update(conv_c(h, src_c, dst_c, zeros_c), deg_c, W2, b2r)
    h = _tc_conv_update(conv_d(h, src_d, dst_d, zeros_d), deg_d, W3, b3r)
    h = _tc_conv_update(conv_c(h, src_c, dst_c, zeros_c), deg_c, W4, b4r)
    out = _tc_conv_head(conv_c(h, src_c, dst_c, zeros_c), deg_c, W4, b4r,
                        Wc, bc)

    out = out[:_N_NODES]
    return (out[None, :, : _OUT // 2], out[None, :, _OUT // 2:])


# speedup vs baseline: 1.1703x; 1.0426x over previous
import functools

import jax
import jax.numpy as jnp
from jax import lax
from jax.experimental import pallas as pl
from jax.experimental.pallas import tpu as pltpu
from jax.experimental.pallas import tpu_sc as plsc

_N_NODES = 10000
_N_PAD = 10240
_D = 128
_OUT = 64
_NC = 2
_NS = 16
_NW = _NC * _NS
_TILE_ROWS = _N_PAD // _NS
_ROW_BLOCK = 1024


def _sc_mesh():
    return plsc.VectorSubcoreMesh(core_axis_name="c", subcore_axis_name="s",
                                  num_cores=_NC, num_subcores=_NS)


def _make_sc_conv(chunks_per_tile, chunk):
    @functools.partial(
        pl.kernel,
        out_type=jax.ShapeDtypeStruct((_NC, _N_PAD, _D), jnp.float32),
        mesh=_sc_mesh(),
        scratch_types=[
            pltpu.VMEM((chunks_per_tile, chunk), jnp.int32),
            pltpu.VMEM((chunks_per_tile, chunk), jnp.int32),
            pltpu.VMEM((chunk, _D), jnp.float32),
            pltpu.VMEM((chunk, _D), jnp.float32),
            pltpu.VMEM_SHARED((_N_PAD, _D), jnp.float32),
            pltpu.SemaphoreType.DMA,
            pltpu.SemaphoreType.DMA,
        ],
    )
    def conv(h_hbm, src_hbm, dst_hbm, zeros_hbm, out_hbm, src_v, dst_v, msg0,
             msg1, acc_sh, sem0, sem1):
        c = lax.axis_index("c")
        s = lax.axis_index("s")
        w = c * _NS + s
        pltpu.sync_copy(src_hbm.at[pl.ds(w * chunks_per_tile, chunks_per_tile)],
                        src_v)
        pltpu.sync_copy(dst_hbm.at[pl.ds(w * chunks_per_tile, chunks_per_tile)],
                        dst_v)
        pltpu.sync_copy(zeros_hbm, msg0)
        for k in range(_TILE_ROWS // chunk):
            pltpu.sync_copy(
                msg0, acc_sh.at[pl.ds(s * _TILE_ROWS + k * chunk, chunk)])
        pltpu.async_copy(h_hbm.at[src_v.at[0]], msg0, sem0)
        plsc.subcore_barrier()

        def wait_dma(buf, sem):
            pltpu.make_async_copy(zeros_hbm, buf, sem).wait()

        @pl.loop(0, chunks_per_tile, step=2)
        def _(j):
            wait_dma(msg0, sem0)
            pltpu.async_copy(h_hbm.at[src_v.at[j + 1]], msg1, sem1)
            pltpu.sync_copy(msg0, acc_sh.at[dst_v.at[j]], add=True)
            wait_dma(msg1, sem1)

            @pl.when(j + 2 < chunks_per_tile)
            def _():
                pltpu.async_copy(h_hbm.at[src_v.at[j + 2]], msg0, sem0)

            pltpu.sync_copy(msg1, acc_sh.at[dst_v.at[j + 1]], add=True)

        plsc.subcore_barrier()
        n_out = _TILE_ROWS // chunk
        for k in range(n_out):
            buf, sem = (msg0, sem0) if k % 2 == 0 else (msg1, sem1)
            if k >= 2:
                pltpu.make_async_copy(zeros_hbm, buf, sem).wait()
            base = s * _TILE_ROWS + k * chunk
            pltpu.sync_copy(acc_sh.at[pl.ds(base, chunk)], buf)
            pltpu.async_copy(buf, out_hbm.at[c, pl.ds(base, chunk)], sem)
        for k in range(max(0, n_out - 2), n_out):
            buf, sem = (msg0, sem0) if k % 2 == 0 else (msg1, sem1)
            pltpu.make_async_copy(zeros_hbm, buf, sem).wait()

    return conv


def _make_sc_deg(chunks_per_tile, chunk, width=_D):
    @functools.partial(
        pl.kernel,
        out_type=jax.ShapeDtypeStruct((_NC, _N_PAD, width), jnp.float32),
        mesh=_sc_mesh(),
        scratch_types=[
            pltpu.VMEM((chunks_per_tile, chunk), jnp.int32),
            pltpu.VMEM((chunk, width), jnp.float32),
            pltpu.VMEM_SHARED((_N_PAD, width), jnp.float32),
            pltpu.SemaphoreType.DMA,
        ],
    )
    def deg(dst_hbm, ones_hbm, zeros_hbm, out_hbm, dst_v, buf_v, deg_sh, sem):
        c = lax.axis_index("c")
        s = lax.axis_index("s")
        w = c * _NS + s
        pltpu.sync_copy(dst_hbm.at[pl.ds(w * chunks_per_tile, chunks_per_tile)],
                        dst_v)
        pltpu.sync_copy(zeros_hbm, buf_v)
        for k in range(_TILE_ROWS // chunk):
            pltpu.sync_copy(
                buf_v, deg_sh.at[pl.ds(s * _TILE_ROWS + k * chunk, chunk)])
        pltpu.sync_copy(ones_hbm, buf_v)
        plsc.subcore_barrier()

        @pl.loop(0, chunks_per_tile)
        def _(j):
            pltpu.async_copy(buf_v, deg_sh.at[dst_v.at[j]], sem, add=True)

        @pl.loop(0, chunks_per_tile)
        def _(j):
            pltpu.make_async_copy(zeros_hbm, buf_v, sem).wait()

        plsc.subcore_barrier()
        for k in range(_TILE_ROWS // chunk):
            base = s * _TILE_ROWS + k * chunk
            pltpu.sync_copy(deg_sh.at[pl.ds(base, chunk)], buf_v)
            pltpu.sync_copy(buf_v, out_hbm.at[c, pl.ds(base, chunk)])

    return deg


def _tc_prep_c(degparts_c, Wl0, bl0, Wl1, bl1, Wo, bo):
    n_blocks = _N_PAD // _ROW_BLOCK
    hp = jax.lax.Precision.HIGHEST

    def body(dgc_ref, w0_ref, b0_ref, w1_ref, b1_ref, wo_ref, bo_ref,
             invc_ref, wc_ref, bc_ref):
        dc = dgc_ref[0, :, 0:1] + dgc_ref[1, :, 0:1]
        invc_ref[...] = 1.0 / jnp.maximum(dc, 1.0)

        @pl.when(pl.program_id(0) == 0)
        def _():
            t = jax.lax.dot_general(w0_ref[...], w1_ref[...],
                                    (((1,), (0,)), ((), ())), precision=hp,
                                    preferred_element_type=jnp.float32)
            wc_ref[...] = jax.lax.dot_general(t, wo_ref[...],
                                              (((1,), (0,)), ((), ())),
                                              precision=hp,
                                              preferred_element_type=jnp.float32)
            u = jax.lax.dot_general(b0_ref[...], w1_ref[...],
                                    (((1,), (0,)), ((), ())), precision=hp,
                                    preferred_element_type=jnp.float32)
            u = u + b1_ref[...]
            bc_ref[...] = jax.lax.dot_general(u, wo_ref[...],
                                              (((1,), (0,)), ((), ())),
                                              precision=hp,
                                              preferred_element_type=jnp.float32
                                              ) + bo_ref[...]

    return pl.pallas_call(
        body,
        grid=(n_blocks,),
        in_specs=[
            pl.BlockSpec((_NC, _ROW_BLOCK, _D), lambda i: (0, i, 0)),
            pl.BlockSpec((_D, _D), lambda i: (0, 0)),
            pl.BlockSpec((1, _D), lambda i: (0, 0)),
            pl.BlockSpec((_D, _D), lambda i: (0, 0)),
            pl.BlockSpec((1, _D), lambda i: (0, 0)),
            pl.BlockSpec((_D, _OUT), lambda i: (0, 0)),
            pl.BlockSpec((1, _OUT), lambda i: (0, 0)),
        ],
        out_specs=[
            pl.BlockSpec((_ROW_BLOCK, 1), lambda i: (i, 0)),
            pl.BlockSpec((_D, _OUT), lambda i: (0, 0)),
            pl.BlockSpec((1, _OUT), lambda i: (0, 0)),
        ],
        out_shape=[
            jax.ShapeDtypeStruct((_N_PAD, 1), jnp.float32),
            jax.ShapeDtypeStruct((_D, _OUT), jnp.float32),
            jax.ShapeDtypeStruct((1, _OUT), jnp.float32),
        ],
    )(degparts_c, Wl0, bl0.reshape(1, _D), Wl1, bl1.reshape(1, _D), Wo,
      bo.reshape(1, _OUT))


def _tc_inv_deg(degparts):
    n_blocks = _N_PAD // _ROW_BLOCK

    def body(dg_ref, o_ref):
        d = dg_ref[0, :, 0:1] + dg_ref[1, :, 0:1]
        o_ref[...] = 1.0 / jnp.maximum(d, 1.0)

    return pl.pallas_call(
        body,
        grid=(n_blocks,),
        in_specs=[pl.BlockSpec((_NC, _ROW_BLOCK, _D), lambda i: (0, i, 0))],
        out_specs=pl.BlockSpec((_ROW_BLOCK, 1), lambda i: (i, 0)),
        out_shape=jax.ShapeDtypeStruct((_N_PAD, 1), jnp.float32),
    )(degparts)


def _tc_conv_update(parts, inv_deg, W, b):
    n_blocks = _N_PAD // _ROW_BLOCK

    def body(p_ref, inv_ref, w_ref, b_ref, o_ref):
        p = (p_ref[0] + p_ref[1]) * inv_ref[...]
        h = jax.lax.dot_general(p, w_ref[...], (((1,), (0,)), ((), ())),
                                precision=jax.lax.Precision.DEFAULT,
                                preferred_element_type=jnp.float32)
        o_ref[...] = jnp.maximum(h + b_ref[...], 0.0)

    return pl.pallas_call(
        body,
        grid=(n_blocks,),
        in_specs=[
            pl.BlockSpec((_NC, _ROW_BLOCK, _D), lambda i: (0, i, 0)),
            pl.BlockSpec((_ROW_BLOCK, 1), lambda i: (i, 0)),
            pl.BlockSpec((_D, _D), lambda i: (0, 0)),
            pl.BlockSpec((1, _D), lambda i: (0, 0)),
        ],
        out_specs=pl.BlockSpec((_ROW_BLOCK, _D), lambda i: (i, 0)),
        out_shape=jax.ShapeDtypeStruct((_N_PAD, _D), jnp.float32),
    )(parts, inv_deg, W, b)


def _tc_conv_head(parts, inv_deg, W, b, Wc, bc):
    n_blocks = _N_PAD // _ROW_BLOCK

    def body(p_ref, inv_ref, w_ref, b_ref, wc_ref, bc_ref, o_ref):
        p = (p_ref[0] + p_ref[1]) * inv_ref[...]
        h = jax.lax.dot_general(p, w_ref[...], (((1,), (0,)), ((), ())),
                                precision=jax.lax.Precision.DEFAULT,
                                preferred_element_type=jnp.float32)
        t = jnp.maximum(h + b_ref[...], 0.0)
        o = jax.lax.dot_general(t, wc_ref[...], (((1,), (0,)), ((), ())),
                                precision=jax.lax.Precision.DEFAULT,
                                preferred_element_type=jnp.float32)
        o_ref[...] = o + bc_ref[...]

    return pl.pallas_call(
        body,
        grid=(n_blocks,),
        in_specs=[
            pl.BlockSpec((_NC, _ROW_BLOCK, _D), lambda i: (0, i, 0)),
            pl.BlockSpec((_ROW_BLOCK, 1), lambda i: (i, 0)),
            pl.BlockSpec((_D, _D), lambda i: (0, 0)),
            pl.BlockSpec((1, _D), lambda i: (0, 0)),
            pl.BlockSpec((_D, _OUT), lambda i: (0, 0)),
            pl.BlockSpec((1, _OUT), lambda i: (0, 0)),
        ],
        out_specs=pl.BlockSpec((_ROW_BLOCK, _OUT), lambda i: (i, 0)),
        out_shape=jax.ShapeDtypeStruct((_N_PAD, _OUT), jnp.float32),
    )(parts, inv_deg, W, b, Wc, bc)


def _prep_edges(ei, chunks_per_tile, chunk):
    e = ei.shape[1]
    e_tot = _NW * chunks_per_tile * chunk
    n_pad = e_tot - e
    fill = jnp.arange(n_pad, dtype=jnp.int32)
    pad_src = fill % _N_NODES
    pad_dst = _N_NODES + fill % (_N_PAD - _N_NODES)
    src = jnp.concatenate([ei[0], pad_src]).reshape(_NW * chunks_per_tile,
                                                    chunk)
    dst = jnp.concatenate([ei[1], pad_dst]).reshape(_NW * chunks_per_tile,
                                                    chunk)
    return src, dst


def kernel(x, edge_index_connections, edge_index_destinations, W1, b1, W2, b2,
           W3, b3, W4, b4, Wl0, bl0, Wl1, bl1, Wo, bo):
    cpt_c, chk_c = 40, 128
    cpt_d, chk_d = 32, 80
    src_c, dst_c = _prep_edges(edge_index_connections, cpt_c, chk_c)
    src_d, dst_d = _prep_edges(edge_index_destinations, cpt_d, chk_d)

    x_pad = jnp.concatenate(
        [x[0], jnp.zeros((_N_PAD - _N_NODES, _D), jnp.float32)], axis=0)
    zeros_c = jnp.zeros((chk_c, _D), jnp.float32)
    zeros_d = jnp.zeros((chk_d, _D), jnp.float32)
    ones_c = jnp.ones((chk_c, _D), jnp.float32)
    ones_d = jnp.ones((chk_d, _D), jnp.float32)

    degparts_c = _make_sc_deg(cpt_c, chk_c)(dst_c, ones_c, zeros_c)
    degparts_d = _make_sc_deg(cpt_d, chk_d)(dst_d, ones_d, zeros_d)
    deg_c, Wc, bc = _tc_prep_c(degparts_c, Wl0, bl0, Wl1, bl1, Wo, bo)
    deg_d = _tc_inv_deg(degparts_d)

    conv_c = _make_sc_conv(cpt_c, chk_c)
    conv_d = _make_sc_conv(cpt_d, chk_d)

    b1r, b2r = b1.reshape(1, _D), b2.reshape(1, _D)
    b3r, b4r = b3.reshape(1, _D), b4.reshape(1, _D)

    h = _tc_conv_update(conv_c(x_pad, src_c, dst_c, zeros_c), deg_c, W1, b1r)
    h = _tc_conv_update(conv_c(h, src_c, dst_c, zeros_c), deg_c, W2, b2r)
    h = _tc_conv_update(conv_c(h, src_c, dst_c, zeros_c), deg_c, W2, b2r)
    h = _tc_conv_update(conv_d(h, src_d, dst_d, zeros_d), deg_d, W3, b3r)
    h = _tc_conv_update(conv_c(h, src_c, dst_c, zeros_c), deg_c, W4, b4r)
    out = _tc_conv_head(conv_c(h, src_c, dst_c, zeros_c), deg_c, W4, b4r,
                        Wc, bc)

    out = out[:_N_NODES]
    return (out[None, :, : _OUT // 2], out[None, :, _OUT // 2:])


# speedup vs baseline: 1.1943x; 1.0205x over previous
import functools

import jax
import jax.numpy as jnp
from jax import lax
from jax.experimental import pallas as pl
from jax.experimental.pallas import tpu as pltpu
from jax.experimental.pallas import tpu_sc as plsc

_N_NODES = 10000
_N_PAD = 10240
_D = 128
_OUT = 64
_NC = 2
_NS = 16
_NW = _NC * _NS
_TILE_ROWS = _N_PAD // _NS
_ROW_BLOCK = 2048


def _sc_mesh():
    return plsc.VectorSubcoreMesh(core_axis_name="c", subcore_axis_name="s",
                                  num_cores=_NC, num_subcores=_NS)


def _make_sc_conv(chunks_per_tile, chunk):
    @functools.partial(
        pl.kernel,
        out_type=jax.ShapeDtypeStruct((_NC, _N_PAD, _D), jnp.float32),
        mesh=_sc_mesh(),
        scratch_types=[
            pltpu.VMEM((chunks_per_tile, chunk), jnp.int32),
            pltpu.VMEM((chunks_per_tile, chunk), jnp.int32),
            pltpu.VMEM((chunk, _D), jnp.float32),
            pltpu.VMEM((chunk, _D), jnp.float32),
            pltpu.VMEM_SHARED((_N_PAD, _D), jnp.float32),
            pltpu.SemaphoreType.DMA,
            pltpu.SemaphoreType.DMA,
        ],
    )
    def conv(h_hbm, src_hbm, dst_hbm, zeros_hbm, out_hbm, src_v, dst_v, msg0,
             msg1, acc_sh, sem0, sem1):
        c = lax.axis_index("c")
        s = lax.axis_index("s")
        w = c * _NS + s
        pltpu.sync_copy(src_hbm.at[pl.ds(w * chunks_per_tile, chunks_per_tile)],
                        src_v)
        pltpu.sync_copy(dst_hbm.at[pl.ds(w * chunks_per_tile, chunks_per_tile)],
                        dst_v)
        pltpu.sync_copy(zeros_hbm, msg0)
        for k in range(_TILE_ROWS // chunk):
            pltpu.sync_copy(
                msg0, acc_sh.at[pl.ds(s * _TILE_ROWS + k * chunk, chunk)])
        pltpu.async_copy(h_hbm.at[src_v.at[0]], msg0, sem0)
        plsc.subcore_barrier()

        def wait_dma(buf, sem):
            pltpu.make_async_copy(zeros_hbm, buf, sem).wait()

        @pl.loop(0, chunks_per_tile, step=2)
        def _(j):
            wait_dma(msg0, sem0)
            pltpu.async_copy(h_hbm.at[src_v.at[j + 1]], msg1, sem1)
            pltpu.sync_copy(msg0, acc_sh.at[dst_v.at[j]], add=True)
            wait_dma(msg1, sem1)

            @pl.when(j + 2 < chunks_per_tile)
            def _():
                pltpu.async_copy(h_hbm.at[src_v.at[j + 2]], msg0, sem0)

            pltpu.sync_copy(msg1, acc_sh.at[dst_v.at[j + 1]], add=True)

        plsc.subcore_barrier()
        n_out = _TILE_ROWS // chunk
        for k in range(n_out):
            buf, sem = (msg0, sem0) if k % 2 == 0 else (msg1, sem1)
            if k >= 2:
                pltpu.make_async_copy(zeros_hbm, buf, sem).wait()
            base = s * _TILE_ROWS + k * chunk
            pltpu.sync_copy(acc_sh.at[pl.ds(base, chunk)], buf)
            pltpu.async_copy(buf, out_hbm.at[c, pl.ds(base, chunk)], sem)
        for k in range(max(0, n_out - 2), n_out):
            buf, sem = (msg0, sem0) if k % 2 == 0 else (msg1, sem1)
            pltpu.make_async_copy(zeros_hbm, buf, sem).wait()

    return conv


def _make_sc_deg(chunks_per_tile, chunk, width=_D):
    @functools.partial(
        pl.kernel,
        out_type=jax.ShapeDtypeStruct((_NC, _N_PAD, width), jnp.float32),
        mesh=_sc_mesh(),
        scratch_types=[
            pltpu.VMEM((chunks_per_tile, chunk), jnp.int32),
            pltpu.VMEM((chunk, width), jnp.float32),
            pltpu.VMEM_SHARED((_N_PAD, width), jnp.float32),
            pltpu.SemaphoreType.DMA,
        ],
    )
    def deg(dst_hbm, ones_hbm, zeros_hbm, out_hbm, dst_v, buf_v, deg_sh, sem):
        c = lax.axis_index("c")
        s = lax.axis_index("s")
        w = c * _NS + s
        pltpu.sync_copy(dst_hbm.at[pl.ds(w * chunks_per_tile, chunks_per_tile)],
                        dst_v)
        pltpu.sync_copy(zeros_hbm, buf_v)
        for k in range(_TILE_ROWS // chunk):
            pltpu.sync_copy(
                buf_v, deg_sh.at[pl.ds(s * _TILE_ROWS + k * chunk, chunk)])
        pltpu.sync_copy(ones_hbm, buf_v)
        plsc.subcore_barrier()

        @pl.loop(0, chunks_per_tile)
        def _(j):
            pltpu.async_copy(buf_v, deg_sh.at[dst_v.at[j]], sem, add=True)

        @pl.loop(0, chunks_per_tile)
        def _(j):
            pltpu.make_async_copy(zeros_hbm, buf_v, sem).wait()

        plsc.subcore_barrier()
        for k in range(_TILE_ROWS // chunk):
            base = s * _TILE_ROWS + k * chunk
            pltpu.sync_copy(deg_sh.at[pl.ds(base, chunk)], buf_v)
            pltpu.sync_copy(buf_v, out_hbm.at[c, pl.ds(base, chunk)])

    return deg


def _tc_prep_c(degparts_c, Wl0, bl0, Wl1, bl1, Wo, bo):
    n_blocks = _N_PAD // _ROW_BLOCK
    hp = jax.lax.Precision.HIGHEST

    def body(dgc_ref, w0_ref, b0_ref, w1_ref, b1_ref, wo_ref, bo_ref,
             invc_ref, wc_ref, bc_ref):
        dc = dgc_ref[0, :, 0:1] + dgc_ref[1, :, 0:1]
        invc_ref[...] = 1.0 / jnp.maximum(dc, 1.0)

        @pl.when(pl.program_id(0) == 0)
        def _():
            t = jax.lax.dot_general(w0_ref[...], w1_ref[...],
                                    (((1,), (0,)), ((), ())), precision=hp,
                                    preferred_element_type=jnp.float32)
            wc_ref[...] = jax.lax.dot_general(t, wo_ref[...],
                                              (((1,), (0,)), ((), ())),
                                              precision=hp,
                                              preferred_element_type=jnp.float32)
            u = jax.lax.dot_general(b0_ref[...], w1_ref[...],
                                    (((1,), (0,)), ((), ())), precision=hp,
                                    preferred_element_type=jnp.float32)
            u = u + b1_ref[...]
            bc_ref[...] = jax.lax.dot_general(u, wo_ref[...],
                                              (((1,), (0,)), ((), ())),
                                              precision=hp,
                                              preferred_element_type=jnp.float32
                                              ) + bo_ref[...]

    return pl.pallas_call(
        body,
        grid=(n_blocks,),
        in_specs=[
            pl.BlockSpec((_NC, _ROW_BLOCK, _D), lambda i: (0, i, 0)),
            pl.BlockSpec((_D, _D), lambda i: (0, 0)),
            pl.BlockSpec((1, _D), lambda i: (0, 0)),
            pl.BlockSpec((_D, _D), lambda i: (0, 0)),
            pl.BlockSpec((1, _D), lambda i: (0, 0)),
            pl.BlockSpec((_D, _OUT), lambda i: (0, 0)),
            pl.BlockSpec((1, _OUT), lambda i: (0, 0)),
        ],
        out_specs=[
            pl.BlockSpec((_ROW_BLOCK, 1), lambda i: (i, 0)),
            pl.BlockSpec((_D, _OUT), lambda i: (0, 0)),
            pl.BlockSpec((1, _OUT), lambda i: (0, 0)),
        ],
        out_shape=[
            jax.ShapeDtypeStruct((_N_PAD, 1), jnp.float32),
            jax.ShapeDtypeStruct((_D, _OUT), jnp.float32),
            jax.ShapeDtypeStruct((1, _OUT), jnp.float32),
        ],
    )(degparts_c, Wl0, bl0.reshape(1, _D), Wl1, bl1.reshape(1, _D), Wo,
      bo.reshape(1, _OUT))


def _tc_inv_deg(degparts):
    n_blocks = _N_PAD // _ROW_BLOCK

    def body(dg_ref, o_ref):
        d = dg_ref[0, :, 0:1] + dg_ref[1, :, 0:1]
        o_ref[...] = 1.0 / jnp.maximum(d, 1.0)

    return pl.pallas_call(
        body,
        grid=(n_blocks,),
        in_specs=[pl.BlockSpec((_NC, _ROW_BLOCK, _D), lambda i: (0, i, 0))],
        out_specs=pl.BlockSpec((_ROW_BLOCK, 1), lambda i: (i, 0)),
        out_shape=jax.ShapeDtypeStruct((_N_PAD, 1), jnp.float32),
    )(degparts)


def _tc_conv_update(parts, inv_deg, W, b):
    n_blocks = _N_PAD // _ROW_BLOCK

    def body(p_ref, inv_ref, w_ref, b_ref, o_ref):
        p = (p_ref[0] + p_ref[1]) * inv_ref[...]
        h = jax.lax.dot_general(p, w_ref[...], (((1,), (0,)), ((), ())),
                                precision=jax.lax.Precision.DEFAULT,
                                preferred_element_type=jnp.float32)
        o_ref[...] = jnp.maximum(h + b_ref[...], 0.0)

    return pl.pallas_call(
        body,
        grid=(n_blocks,),
        in_specs=[
            pl.BlockSpec((_NC, _ROW_BLOCK, _D), lambda i: (0, i, 0)),
            pl.BlockSpec((_ROW_BLOCK, 1), lambda i: (i, 0)),
            pl.BlockSpec((_D, _D), lambda i: (0, 0)),
            pl.BlockSpec((1, _D), lambda i: (0, 0)),
        ],
        out_specs=pl.BlockSpec((_ROW_BLOCK, _D), lambda i: (i, 0)),
        out_shape=jax.ShapeDtypeStruct((_N_PAD, _D), jnp.float32),
    )(parts, inv_deg, W, b)


def _tc_conv_head(parts, inv_deg, W, b, Wc, bc):
    n_blocks = _N_PAD // _ROW_BLOCK

    def body(p_ref, inv_ref, w_ref, b_ref, wc_ref, bc_ref, o_ref):
        p = (p_ref[0] + p_ref[1]) * inv_ref[...]
        h = jax.lax.dot_general(p, w_ref[...], (((1,), (0,)), ((), ())),
                                precision=jax.lax.Precision.DEFAULT,
                                preferred_element_type=jnp.float32)
        t = jnp.maximum(h + b_ref[...], 0.0)
        o = jax.lax.dot_general(t, wc_ref[...], (((1,), (0,)), ((), ())),
                                precision=jax.lax.Precision.DEFAULT,
                                preferred_element_type=jnp.float32)
        o_ref[...] = o + bc_ref[...]

    return pl.pallas_call(
        body,
        grid=(n_blocks,),
        in_specs=[
            pl.BlockSpec((_NC, _ROW_BLOCK, _D), lambda i: (0, i, 0)),
            pl.BlockSpec((_ROW_BLOCK, 1), lambda i: (i, 0)),
            pl.BlockSpec((_D, _D), lambda i: (0, 0)),
            pl.BlockSpec((1, _D), lambda i: (0, 0)),
            pl.BlockSpec((_D, _OUT), lambda i: (0, 0)),
            pl.BlockSpec((1, _OUT), lambda i: (0, 0)),
        ],
        out_specs=pl.BlockSpec((_ROW_BLOCK, _OUT), lambda i: (i, 0)),
        out_shape=jax.ShapeDtypeStruct((_N_PAD, _OUT), jnp.float32),
    )(parts, inv_deg, W, b, Wc, bc)


def _prep_edges(ei, chunks_per_tile, chunk):
    e = ei.shape[1]
    e_tot = _NW * chunks_per_tile * chunk
    n_pad = e_tot - e
    fill = jnp.arange(n_pad, dtype=jnp.int32)
    pad_src = fill % _N_NODES
    pad_dst = _N_NODES + fill % (_N_PAD - _N_NODES)
    src = jnp.concatenate([ei[0], pad_src]).reshape(_NW * chunks_per_tile,
                                                    chunk)
    dst = jnp.concatenate([ei[1], pad_dst]).reshape(_NW * chunks_per_tile,
                                                    chunk)
    return src, dst


def kernel(x, edge_index_connections, edge_index_destinations, W1, b1, W2, b2,
           W3, b3, W4, b4, Wl0, bl0, Wl1, bl1, Wo, bo):
    cpt_c, chk_c = 40, 128
    cpt_d, chk_d = 32, 80
    src_c, dst_c = _prep_edges(edge_index_connections, cpt_c, chk_c)
    src_d, dst_d = _prep_edges(edge_index_destinations, cpt_d, chk_d)

    x_pad = jnp.concatenate(
        [x[0], jnp.zeros((_N_PAD - _N_NODES, _D), jnp.float32)], axis=0)
    zeros_c = jnp.zeros((chk_c, _D), jnp.float32)
    zeros_d = jnp.zeros((chk_d, _D), jnp.float32)
    ones_c = jnp.ones((chk_c, _D), jnp.float32)
    ones_d = jnp.ones((chk_d, _D), jnp.float32)

    degparts_c = _make_sc_deg(cpt_c, chk_c)(dst_c, ones_c, zeros_c)
    degparts_d = _make_sc_deg(cpt_d, chk_d)(dst_d, ones_d, zeros_d)
    deg_c, Wc, bc = _tc_prep_c(degparts_c, Wl0, bl0, Wl1, bl1, Wo, bo)
    deg_d = _tc_inv_deg(degparts_d)

    conv_c = _make_sc_conv(cpt_c, chk_c)
    conv_d = _make_sc_conv(cpt_d, chk_d)

    b1r, b2r = b1.reshape(1, _D), b2.reshape(1, _D)
    b3r, b4r = b3.reshape(1, _D), b4.reshape(1, _D)

    h = _tc_conv_update(conv_c(x_pad, src_c, dst_c, zeros_c), deg_c, W1, b1r)
    h = _tc_conv_update(conv_c(h, src_c, dst_c, zeros_c), deg_c, W2, b2r)
    h = _tc_conv_update(conv_c(h, src_c, dst_c, zeros_c), deg_c, W2, b2r)
    h = _tc_conv_update(conv_d(h, src_d, dst_d, zeros_d), deg_d, W3, b3r)
    h = _tc_conv_update(conv_c(h, src_c, dst_c, zeros_c), deg_c, W4, b4r)
    out = _tc_conv_head(conv_c(h, src_c, dst_c, zeros_c), deg_c, W4, b4r,
                        Wc, bc)

    out = out[:_N_NODES]
    return (out[None, :, : _OUT // 2], out[None, :, _OUT // 2:])


# speedup vs baseline: 1.2095x; 1.0128x over previous
import functools

import jax
import jax.numpy as jnp
from jax import lax
from jax.experimental import pallas as pl
from jax.experimental.pallas import tpu as pltpu
from jax.experimental.pallas import tpu_sc as plsc

_N_NODES = 10000
_N_PAD = 10240
_D = 128
_OUT = 64
_NC = 2
_NS = 16
_NW = _NC * _NS
_TILE_ROWS = _N_PAD // _NS
_ROW_BLOCK = 5120


def _sc_mesh():
    return plsc.VectorSubcoreMesh(core_axis_name="c", subcore_axis_name="s",
                                  num_cores=_NC, num_subcores=_NS)


def _make_sc_conv(chunks_per_tile, chunk):
    @functools.partial(
        pl.kernel,
        out_type=jax.ShapeDtypeStruct((_NC, _N_PAD, _D), jnp.float32),
        mesh=_sc_mesh(),
        scratch_types=[
            pltpu.VMEM((chunks_per_tile, chunk), jnp.int32),
            pltpu.VMEM((chunks_per_tile, chunk), jnp.int32),
            pltpu.VMEM((chunk, _D), jnp.float32),
            pltpu.VMEM((chunk, _D), jnp.float32),
            pltpu.VMEM_SHARED((_N_PAD, _D), jnp.float32),
            pltpu.SemaphoreType.DMA,
            pltpu.SemaphoreType.DMA,
        ],
    )
    def conv(h_hbm, src_hbm, dst_hbm, zeros_hbm, out_hbm, src_v, dst_v, msg0,
             msg1, acc_sh, sem0, sem1):
        c = lax.axis_index("c")
        s = lax.axis_index("s")
        w = c * _NS + s
        pltpu.sync_copy(src_hbm.at[pl.ds(w * chunks_per_tile, chunks_per_tile)],
                        src_v)
        pltpu.sync_copy(dst_hbm.at[pl.ds(w * chunks_per_tile, chunks_per_tile)],
                        dst_v)
        pltpu.sync_copy(zeros_hbm, msg0)
        for k in range(_TILE_ROWS // chunk):
            pltpu.sync_copy(
                msg0, acc_sh.at[pl.ds(s * _TILE_ROWS + k * chunk, chunk)])
        pltpu.async_copy(h_hbm.at[src_v.at[0]], msg0, sem0)
        plsc.subcore_barrier()

        def wait_dma(buf, sem):
            pltpu.make_async_copy(zeros_hbm, buf, sem).wait()

        @pl.loop(0, chunks_per_tile, step=2)
        def _(j):
            wait_dma(msg0, sem0)
            pltpu.async_copy(h_hbm.at[src_v.at[j + 1]], msg1, sem1)
            pltpu.sync_copy(msg0, acc_sh.at[dst_v.at[j]], add=True)
            wait_dma(msg1, sem1)

            @pl.when(j + 2 < chunks_per_tile)
            def _():
                pltpu.async_copy(h_hbm.at[src_v.at[j + 2]], msg0, sem0)

            pltpu.sync_copy(msg1, acc_sh.at[dst_v.at[j + 1]], add=True)

        plsc.subcore_barrier()
        n_out = _TILE_ROWS // chunk
        for k in range(n_out):
            buf, sem = (msg0, sem0) if k % 2 == 0 else (msg1, sem1)
            if k >= 2:
                pltpu.make_async_copy(zeros_hbm, buf, sem).wait()
            base = s * _TILE_ROWS + k * chunk
            pltpu.sync_copy(acc_sh.at[pl.ds(base, chunk)], buf)
            pltpu.async_copy(buf, out_hbm.at[c, pl.ds(base, chunk)], sem)
        for k in range(max(0, n_out - 2), n_out):
            buf, sem = (msg0, sem0) if k % 2 == 0 else (msg1, sem1)
            pltpu.make_async_copy(zeros_hbm, buf, sem).wait()

    return conv


def _make_sc_deg(chunks_per_tile, chunk, width=_D):
    @functools.partial(
        pl.kernel,
        out_type=jax.ShapeDtypeStruct((_NC, _N_PAD, width), jnp.float32),
        mesh=_sc_mesh(),
        scratch_types=[
            pltpu.VMEM((chunks_per_tile, chunk), jnp.int32),
            pltpu.VMEM((chunk, width), jnp.float32),
            pltpu.VMEM_SHARED((_N_PAD, width), jnp.float32),
            pltpu.SemaphoreType.DMA,
        ],
    )
    def deg(dst_hbm, ones_hbm, zeros_hbm, out_hbm, dst_v, buf_v, deg_sh, sem):
        c = lax.axis_index("c")
        s = lax.axis_index("s")
        w = c * _NS + s
        pltpu.sync_copy(dst_hbm.at[pl.ds(w * chunks_per_tile, chunks_per_tile)],
                        dst_v)
        pltpu.sync_copy(zeros_hbm, buf_v)
        for k in range(_TILE_ROWS // chunk):
            pltpu.sync_copy(
                buf_v, deg_sh.at[pl.ds(s * _TILE_ROWS + k * chunk, chunk)])
        pltpu.sync_copy(ones_hbm, buf_v)
        plsc.subcore_barrier()

        @pl.loop(0, chunks_per_tile)
        def _(j):
            pltpu.async_copy(buf_v, deg_sh.at[dst_v.at[j]], sem, add=True)

        @pl.loop(0, chunks_per_tile)
        def _(j):
            pltpu.make_async_copy(zeros_hbm, buf_v, sem).wait()

        plsc.subcore_barrier()
        for k in range(_TILE_ROWS // chunk):
            base = s * _TILE_ROWS + k * chunk
            pltpu.sync_copy(deg_sh.at[pl.ds(base, chunk)], buf_v)
            pltpu.sync_copy(buf_v, out_hbm.at[c, pl.ds(base, chunk)])

    return deg


def _tc_prep_c(degparts_c, Wl0, bl0, Wl1, bl1, Wo, bo):
    n_blocks = _N_PAD // _ROW_BLOCK
    hp = jax.lax.Precision.HIGHEST

    def body(dgc_ref, w0_ref, b0_ref, w1_ref, b1_ref, wo_ref, bo_ref,
             invc_ref, wc_ref, bc_ref):
        dc = dgc_ref[0, :, 0:1] + dgc_ref[1, :, 0:1]
        invc_ref[...] = 1.0 / jnp.maximum(dc, 1.0)

        @pl.when(pl.program_id(0) == 0)
        def _():
            t = jax.lax.dot_general(w0_ref[...], w1_ref[...],
                                    (((1,), (0,)), ((), ())), precision=hp,
                                    preferred_element_type=jnp.float32)
            wc_ref[...] = jax.lax.dot_general(t, wo_ref[...],
                                              (((1,), (0,)), ((), ())),
                                              precision=hp,
                                              preferred_element_type=jnp.float32)
            u = jax.lax.dot_general(b0_ref[...], w1_ref[...],
                                    (((1,), (0,)), ((), ())), precision=hp,
                                    preferred_element_type=jnp.float32)
            u = u + b1_ref[...]
            bc_ref[...] = jax.lax.dot_general(u, wo_ref[...],
                                              (((1,), (0,)), ((), ())),
                                              precision=hp,
                                              preferred_element_type=jnp.float32
                                              ) + bo_ref[...]

    return pl.pallas_call(
        body,
        grid=(n_blocks,),
        in_specs=[
            pl.BlockSpec((_NC, _ROW_BLOCK, _D), lambda i: (0, i, 0)),
            pl.BlockSpec((_D, _D), lambda i: (0, 0)),
            pl.BlockSpec((1, _D), lambda i: (0, 0)),
            pl.BlockSpec((_D, _D), lambda i: (0, 0)),
            pl.BlockSpec((1, _D), lambda i: (0, 0)),
            pl.BlockSpec((_D, _OUT), lambda i: (0, 0)),
            pl.BlockSpec((1, _OUT), lambda i: (0, 0)),
        ],
        out_specs=[
            pl.BlockSpec((_ROW_BLOCK, 1), lambda i: (i, 0)),
            pl.BlockSpec((_D, _OUT), lambda i: (0, 0)),
            pl.BlockSpec((1, _OUT), lambda i: (0, 0)),
        ],
        out_shape=[
            jax.ShapeDtypeStruct((_N_PAD, 1), jnp.float32),
            jax.ShapeDtypeStruct((_D, _OUT), jnp.float32),
            jax.ShapeDtypeStruct((1, _OUT), jnp.float32),
        ],
    )(degparts_c, Wl0, bl0.reshape(1, _D), Wl1, bl1.reshape(1, _D), Wo,
      bo.reshape(1, _OUT))


def _tc_inv_deg(degparts):
    n_blocks = _N_PAD // _ROW_BLOCK

    def body(dg_ref, o_ref):
        d = dg_ref[0, :, 0:1] + dg_ref[1, :, 0:1]
        o_ref[...] = 1.0 / jnp.maximum(d, 1.0)

    return pl.pallas_call(
        body,
        grid=(n_blocks,),
        in_specs=[pl.BlockSpec((_NC, _ROW_BLOCK, _D), lambda i: (0, i, 0))],
        out_specs=pl.BlockSpec((_ROW_BLOCK, 1), lambda i: (i, 0)),
        out_shape=jax.ShapeDtypeStruct((_N_PAD, 1), jnp.float32),
    )(degparts)


def _tc_conv_update(parts, inv_deg, W, b):
    n_blocks = _N_PAD // _ROW_BLOCK

    def body(p_ref, inv_ref, w_ref, b_ref, o_ref):
        p = (p_ref[0] + p_ref[1]) * inv_ref[...]
        h = jax.lax.dot_general(p, w_ref[...], (((1,), (0,)), ((), ())),
                                precision=jax.lax.Precision.DEFAULT,
                                preferred_element_type=jnp.float32)
        o_ref[...] = jnp.maximum(h + b_ref[...], 0.0)

    return pl.pallas_call(
        body,
        grid=(n_blocks,),
        in_specs=[
            pl.BlockSpec((_NC, _ROW_BLOCK, _D), lambda i: (0, i, 0)),
            pl.BlockSpec((_ROW_BLOCK, 1), lambda i: (i, 0)),
            pl.BlockSpec((_D, _D), lambda i: (0, 0)),
            pl.BlockSpec((1, _D), lambda i: (0, 0)),
        ],
        out_specs=pl.BlockSpec((_ROW_BLOCK, _D), lambda i: (i, 0)),
        out_shape=jax.ShapeDtypeStruct((_N_PAD, _D), jnp.float32),
    )(parts, inv_deg, W, b)


def _tc_conv_head(parts, inv_deg, W, b, Wc, bc):
    n_blocks = _N_PAD // _ROW_BLOCK

    def body(p_ref, inv_ref, w_ref, b_ref, wc_ref, bc_ref, o_ref):
        p = (p_ref[0] + p_ref[1]) * inv_ref[...]
        h = jax.lax.dot_general(p, w_ref[...], (((1,), (0,)), ((), ())),
                                precision=jax.lax.Precision.DEFAULT,
                                preferred_element_type=jnp.float32)
        t = jnp.maximum(h + b_ref[...], 0.0)
        o = jax.lax.dot_general(t, wc_ref[...], (((1,), (0,)), ((), ())),
                                precision=jax.lax.Precision.DEFAULT,
                                preferred_element_type=jnp.float32)
        o_ref[...] = o + bc_ref[...]

    return pl.pallas_call(
        body,
        grid=(n_blocks,),
        in_specs=[
            pl.BlockSpec((_NC, _ROW_BLOCK, _D), lambda i: (0, i, 0)),
            pl.BlockSpec((_ROW_BLOCK, 1), lambda i: (i, 0)),
            pl.BlockSpec((_D, _D), lambda i: (0, 0)),
            pl.BlockSpec((1, _D), lambda i: (0, 0)),
            pl.BlockSpec((_D, _OUT), lambda i: (0, 0)),
            pl.BlockSpec((1, _OUT), lambda i: (0, 0)),
        ],
        out_specs=pl.BlockSpec((_ROW_BLOCK, _OUT), lambda i: (i, 0)),
        out_shape=jax.ShapeDtypeStruct((_N_PAD, _OUT), jnp.float32),
    )(parts, inv_deg, W, b, Wc, bc)


def _prep_edges(ei, chunks_per_tile, chunk):
    e = ei.shape[1]
    e_tot = _NW * chunks_per_tile * chunk
    n_pad = e_tot - e
    fill = jnp.arange(n_pad, dtype=jnp.int32)
    pad_src = fill % _N_NODES
    pad_dst = _N_NODES + fill % (_N_PAD - _N_NODES)
    src = jnp.concatenate([ei[0], pad_src]).reshape(_NW * chunks_per_tile,
                                                    chunk)
    dst = jnp.concatenate([ei[1], pad_dst]).reshape(_NW * chunks_per_tile,
                                                    chunk)
    return src, dst


def kernel(x, edge_index_connections, edge_index_destinations, W1, b1, W2, b2,
           W3, b3, W4, b4, Wl0, bl0, Wl1, bl1, Wo, bo):
    cpt_c, chk_c = 40, 128
    cpt_d, chk_d = 32, 80
    src_c, dst_c = _prep_edges(edge_index_connections, cpt_c, chk_c)
    src_d, dst_d = _prep_edges(edge_index_destinations, cpt_d, chk_d)

    x_pad = jnp.concatenate(
        [x[0], jnp.zeros((_N_PAD - _N_NODES, _D), jnp.float32)], axis=0)
    zeros_c = jnp.zeros((chk_c, _D), jnp.float32)
    zeros_d = jnp.zeros((chk_d, _D), jnp.float32)
    ones_c = jnp.ones((chk_c, _D), jnp.float32)
    ones_d = jnp.ones((chk_d, _D), jnp.float32)

    degparts_c = _make_sc_deg(cpt_c, chk_c)(dst_c, ones_c, zeros_c)
    degparts_d = _make_sc_deg(cpt_d, chk_d)(dst_d, ones_d, zeros_d)
    deg_c, Wc, bc = _tc_prep_c(degparts_c, Wl0, bl0, Wl1, bl1, Wo, bo)
    deg_d = _tc_inv_deg(degparts_d)

    conv_c = _make_sc_conv(cpt_c, chk_c)
    conv_d = _make_sc_conv(cpt_d, chk_d)

    b1r, b2r = b1.reshape(1, _D), b2.reshape(1, _D)
    b3r, b4r = b3.reshape(1, _D), b4.reshape(1, _D)

    h = _tc_conv_update(conv_c(x_pad, src_c, dst_c, zeros_c), deg_c, W1, b1r)
    h = _tc_conv_update(conv_c(h, src_c, dst_c, zeros_c), deg_c, W2, b2r)
    h = _tc_conv_update(conv_c(h, src_c, dst_c, zeros_c), deg_c, W2, b2r)
    h = _tc_conv_update(conv_d(h, src_d, dst_d, zeros_d), deg_d, W3, b3r)
    h = _tc_conv_update(conv_c(h, src_c, dst_c, zeros_c), deg_c, W4, b4r)
    out = _tc_conv_head(conv_c(h, src_c, dst_c, zeros_c), deg_c, W4, b4r,
                        Wc, bc)

    out = out[:_N_NODES]
    return (out[None, :, : _OUT // 2], out[None, :, _OUT // 2:])
